# Initial kernel scaffold; baseline (speedup 1.0000x reference)
import jax
import jax.numpy as jnp
from jax.experimental import pallas as pl


def kernel(x, edge_index, edge_attr, A1, b1, root1, bias1, A2, b2, root2, bias2, A3, b3, root3, bias3, Wo1, bo1, Wo2, bo2):
    raise NotImplementedError("write your pallas kernel here")


# speedup vs baseline: 2.0027x; 2.0027x over previous
import functools

import jax
import jax.numpy as jnp
from jax import lax
from jax.experimental import pallas as pl
from jax.experimental.pallas import tpu as pltpu
from jax.experimental.pallas import tpu_sc as plsc

N = 10000
E = 160000
NODE_IN = 128
EDGE_IN = 4
H = 16

NC = 2
NS = 16
NW = NC * NS
EPW = E // NW
CH = 40
NCHUNK = EPW // CH
RPT = N // NS

_mesh = plsc.VectorSubcoreMesh(core_axis_name="c", subcore_axis_name="s")
_sc_params = pltpu.CompilerParams(use_tc_tiling_on_sc=False)


def _wid():
    return lax.axis_index("s") * NC + lax.axis_index("c")


def _make_gather(D):
    @functools.partial(
        pl.kernel,
        out_type=jax.ShapeDtypeStruct((E, D), jnp.float32),
        mesh=_mesh,
        compiler_params=_sc_params,
        scratch_types=[
            pltpu.VMEM((NCHUNK, CH), jnp.int32),
            pltpu.VMEM((CH, D), jnp.float32),
            pltpu.SemaphoreType.DMA,
        ],
    )
    def gather(table_hbm, idx_hbm, out_hbm, idx_v, rows_v, sem):
        w = _wid()
        pltpu.sync_copy(idx_hbm.at[w], idx_v)

        def body(j, carry):
            pltpu.async_copy(table_hbm.at[idx_v.at[j]], rows_v, sem).wait()
            pltpu.sync_copy(rows_v, out_hbm.at[pl.ds(w * EPW + j * CH, CH)])
            return carry

        lax.fori_loop(0, NCHUNK, body, 0)

    return gather


_gather128 = _make_gather(NODE_IN)
_gather16 = _make_gather(H)


def _make_scatter(with_cnt):
    outs = [jax.ShapeDtypeStruct((NC, NS, RPT, H), jnp.float32)]
    scratch = [
        pltpu.VMEM((NCHUNK, CH), jnp.int32),
        pltpu.VMEM((EPW, H), jnp.float32),
        pltpu.VMEM_SHARED((N, H), jnp.float32),
    ]
    if with_cnt:
        outs.append(jax.ShapeDtypeStruct((NC, NS, RPT, H), jnp.float32))
        scratch += [
            pltpu.VMEM((CH, H), jnp.float32),
            pltpu.VMEM_SHARED((N, H), jnp.float32),
        ]

    if with_cnt:
        @functools.partial(pl.kernel, out_type=tuple(outs), mesh=_mesh,
                           compiler_params=_sc_params, scratch_types=scratch)
        def scatter(msg_hbm, dst_hbm, zeros_hbm, ones_hbm, agg_out, cnt_out,
                    dst_v, msg_v, agg_sh, ones_v, cnt_sh):
            cid = lax.axis_index("c")
            sid = lax.axis_index("s")
            w = sid * NC + cid
            pltpu.sync_copy(zeros_hbm, agg_sh.at[pl.ds(sid * RPT, RPT)])
            pltpu.sync_copy(zeros_hbm, cnt_sh.at[pl.ds(sid * RPT, RPT)])
            pltpu.sync_copy(ones_hbm, ones_v)
            pltpu.sync_copy(dst_hbm.at[w], dst_v)
            pltpu.sync_copy(msg_hbm.at[pl.ds(w * EPW, EPW)], msg_v)
            plsc.subcore_barrier()

            def body(j, carry):
                pltpu.sync_copy(msg_v.at[pl.ds(j * CH, CH)],
                                agg_sh.at[dst_v.at[j]], add=True)
                pltpu.sync_copy(ones_v, cnt_sh.at[dst_v.at[j]], add=True)
                return carry

            lax.fori_loop(0, NCHUNK, body, 0)
            plsc.subcore_barrier()
            pltpu.sync_copy(agg_sh.at[pl.ds(sid * RPT, RPT)],
                            agg_out.at[cid, sid])
            pltpu.sync_copy(cnt_sh.at[pl.ds(sid * RPT, RPT)],
                            cnt_out.at[cid, sid])
    else:
        @functools.partial(pl.kernel, out_type=outs[0], mesh=_mesh,
                           compiler_params=_sc_params, scratch_types=scratch)
        def scatter(msg_hbm, dst_hbm, zeros_hbm, agg_out,
                    dst_v, msg_v, agg_sh):
            cid = lax.axis_index("c")
            sid = lax.axis_index("s")
            w = sid * NC + cid
            pltpu.sync_copy(zeros_hbm, agg_sh.at[pl.ds(sid * RPT, RPT)])
            pltpu.sync_copy(dst_hbm.at[w], dst_v)
            pltpu.sync_copy(msg_hbm.at[pl.ds(w * EPW, EPW)], msg_v)
            plsc.subcore_barrier()

            def body(j, carry):
                pltpu.sync_copy(msg_v.at[pl.ds(j * CH, CH)],
                                agg_sh.at[dst_v.at[j]], add=True)
                return carry

            lax.fori_loop(0, NCHUNK, body, 0)
            plsc.subcore_barrier()
            pltpu.sync_copy(agg_sh.at[pl.ds(sid * RPT, RPT)],
                            agg_out.at[cid, sid])

    return scatter


_scatter_cnt = _make_scatter(True)
_scatter = _make_scatter(False)


BE1 = 640
BE2 = 2000


def _msg_body(in_dim, be, attr_ref, xs_ref, Ap_ref, bp_ref, sel_ref, out_ref):
    w = jnp.dot(attr_ref[...], Ap_ref[...],
                preferred_element_type=jnp.float32) + bp_ref[...]
    w = jnp.maximum(w, 0.0)
    xs_t = jnp.broadcast_to(xs_ref[...][:, None, :],
                            (be, H, in_dim)).reshape(be, H * in_dim)
    out_ref[...] = jnp.dot(w * xs_t, sel_ref[...],
                           preferred_element_type=jnp.float32)


def _make_msg(in_dim, be):
    k = in_dim * H
    grid = E // be
    return pl.pallas_call(
        functools.partial(_msg_body, in_dim, be),
        grid=(grid,),
        in_specs=[
            pl.BlockSpec((be, EDGE_IN), lambda i: (i, 0)),
            pl.BlockSpec((be, in_dim), lambda i: (i, 0)),
            pl.BlockSpec((EDGE_IN, k), lambda i: (0, 0)),
            pl.BlockSpec((1, k), lambda i: (0, 0)),
            pl.BlockSpec((k, H), lambda i: (0, 0)),
        ],
        out_specs=pl.BlockSpec((be, H), lambda i: (i, 0)),
        out_shape=jax.ShapeDtypeStruct((E, H), jnp.float32),
    )


_msg1 = _make_msg(NODE_IN, BE1)
_msg23 = _make_msg(H, BE2)


def _update_body(agg_ref, cnt_ref, x_ref, root_ref, bias_ref, out_ref):
    agg = agg_ref[0] + agg_ref[1]
    cnt = jnp.maximum(cnt_ref[0] + cnt_ref[1], 1.0)
    out_ref[...] = jnp.maximum(
        agg / cnt + jnp.dot(x_ref[...], root_ref[...],
                            preferred_element_type=jnp.float32) + bias_ref[...],
        0.0)


def _make_update(in_dim):
    return pl.pallas_call(
        _update_body,
        out_shape=jax.ShapeDtypeStruct((N, H), jnp.float32),
    )


_update128 = _make_update(NODE_IN)
_update16 = _make_update(H)


def _final_body(agg_ref, cnt_ref, x_ref, root_ref, bias_ref,
                Wo1_ref, bo1_ref, Wo2_ref, bo2_ref, out_ref):
    agg = agg_ref[0] + agg_ref[1]
    cnt = jnp.maximum(cnt_ref[0] + cnt_ref[1], 1.0)
    h3 = jnp.maximum(
        agg / cnt + jnp.dot(x_ref[...], root_ref[...],
                            preferred_element_type=jnp.float32) + bias_ref[...],
        0.0)
    h4 = jnp.maximum(jnp.dot(h3, Wo1_ref[...],
                             preferred_element_type=jnp.float32) + bo1_ref[...],
                     0.0)
    out_ref[...] = (jnp.dot(h4, Wo2_ref[...],
                            preferred_element_type=jnp.float32) + bo2_ref[...])


_final = pl.pallas_call(
    _final_body,
    out_shape=jax.ShapeDtypeStruct((N, 1), jnp.float32),
)


def _perm(A, b, in_dim):
    Ap = A.reshape(EDGE_IN, in_dim, H).transpose(0, 2, 1).reshape(EDGE_IN, in_dim * H)
    bp = b.reshape(in_dim, H).T.reshape(1, in_dim * H)
    return Ap, bp


def _sel(in_dim):
    k = in_dim * H
    rows = jnp.arange(k) // in_dim
    return (rows[:, None] == jnp.arange(H)[None, :]).astype(jnp.float32)


def kernel(x, edge_index, edge_attr,
           A1, b1, root1, bias1,
           A2, b2, root2, bias2,
           A3, b3, root3, bias3,
           Wo1, bo1, Wo2, bo2):
    src = edge_index[0].reshape(NW, NCHUNK, CH)
    dst = edge_index[1].reshape(NW, NCHUNK, CH)
    zeros = jnp.zeros((RPT, H), jnp.float32)
    ones = jnp.ones((CH, H), jnp.float32)

    A1p, b1p = _perm(A1, b1, NODE_IN)
    A2p, b2p = _perm(A2, b2, H)
    A3p, b3p = _perm(A3, b3, H)
    sel1 = _sel(NODE_IN)
    sel2 = _sel(H)

    xs = _gather128(x, src)
    msg = _msg1(edge_attr, xs, A1p, b1p, sel1)
    agg, cnt = _scatter_cnt(msg, dst, zeros, ones)
    agg = agg.reshape(NC, N, H)
    cnt = cnt.reshape(NC, N, H)
    h = _update128(agg, cnt, x, root1, bias1.reshape(1, H))

    xs = _gather16(h, src)
    msg = _msg23(edge_attr, xs, A2p, b2p, sel2)
    agg = _scatter(msg, dst, zeros).reshape(NC, N, H)
    h = _update16(agg, cnt, h, root2, bias2.reshape(1, H))

    xs = _gather16(h, src)
    msg = _msg23(edge_attr, xs, A3p, b3p, sel2)
    agg = _scatter(msg, dst, zeros).reshape(NC, N, H)
    out = _final(agg, cnt, h, root3, bias3.reshape(1, H),
                 Wo1, bo1.reshape(1, H), Wo2, bo2.reshape(1, 1))
    return out


# speedup vs baseline: 2.5528x; 1.2747x over previous
import functools

import jax
import jax.numpy as jnp
from jax import lax
from jax.experimental import pallas as pl
from jax.experimental.pallas import tpu as pltpu
from jax.experimental.pallas import tpu_sc as plsc

N = 10000
E = 160000
NODE_IN = 128
EDGE_IN = 4
H = 16

NC = 2
NS = 16
NW = NC * NS
EPW = E // NW
CH = 40
NCHUNK = EPW // CH
RPT = N // NS

_mesh = plsc.VectorSubcoreMesh(core_axis_name="c", subcore_axis_name="s")
_sc_params = pltpu.CompilerParams(use_tc_tiling_on_sc=False)


def _wid():
    return lax.axis_index("s") * NC + lax.axis_index("c")


def _make_gather(D):
    @functools.partial(
        pl.kernel,
        out_type=jax.ShapeDtypeStruct((E, D), jnp.float32),
        mesh=_mesh,
        compiler_params=_sc_params,
        scratch_types=[
            pltpu.VMEM((NCHUNK, CH), jnp.int32),
            pltpu.VMEM((CH, D), jnp.float32),
            pltpu.SemaphoreType.DMA,
        ],
    )
    def gather(table_hbm, idx_hbm, out_hbm, idx_v, rows_v, sem):
        w = _wid()
        pltpu.sync_copy(idx_hbm.at[w], idx_v)

        def body(j, carry):
            pltpu.async_copy(table_hbm.at[idx_v.at[j]], rows_v, sem).wait()
            pltpu.sync_copy(rows_v, out_hbm.at[pl.ds(w * EPW + j * CH, CH)])
            return carry

        lax.fori_loop(0, NCHUNK, body, 0)

    return gather


_gather128 = _make_gather(NODE_IN)
_gather16 = _make_gather(H)


def _make_scatter(with_cnt):
    outs = [jax.ShapeDtypeStruct((NC, NS, RPT, H), jnp.float32)]
    scratch = [
        pltpu.VMEM((NCHUNK, CH), jnp.int32),
        pltpu.VMEM((EPW, H), jnp.float32),
        pltpu.VMEM_SHARED((N, H), jnp.float32),
    ]
    if with_cnt:
        outs.append(jax.ShapeDtypeStruct((NC, NS, RPT, H), jnp.float32))
        scratch += [
            pltpu.VMEM((CH, H), jnp.float32),
            pltpu.VMEM_SHARED((N, H), jnp.float32),
        ]

    if with_cnt:
        @functools.partial(pl.kernel, out_type=tuple(outs), mesh=_mesh,
                           compiler_params=_sc_params, scratch_types=scratch)
        def scatter(msg_hbm, dst_hbm, zeros_hbm, ones_hbm, agg_out, cnt_out,
                    dst_v, msg_v, agg_sh, ones_v, cnt_sh):
            cid = lax.axis_index("c")
            sid = lax.axis_index("s")
            w = sid * NC + cid
            pltpu.sync_copy(zeros_hbm, agg_sh.at[pl.ds(sid * RPT, RPT)])
            pltpu.sync_copy(zeros_hbm, cnt_sh.at[pl.ds(sid * RPT, RPT)])
            pltpu.sync_copy(ones_hbm, ones_v)
            pltpu.sync_copy(dst_hbm.at[w], dst_v)
            pltpu.sync_copy(msg_hbm.at[pl.ds(w * EPW, EPW)], msg_v)
            plsc.subcore_barrier()

            def body(j, carry):
                pltpu.sync_copy(msg_v.at[pl.ds(j * CH, CH)],
                                agg_sh.at[dst_v.at[j]], add=True)
                pltpu.sync_copy(ones_v, cnt_sh.at[dst_v.at[j]], add=True)
                return carry

            lax.fori_loop(0, NCHUNK, body, 0)
            plsc.subcore_barrier()
            pltpu.sync_copy(agg_sh.at[pl.ds(sid * RPT, RPT)],
                            agg_out.at[cid, sid])
            pltpu.sync_copy(cnt_sh.at[pl.ds(sid * RPT, RPT)],
                            cnt_out.at[cid, sid])
    else:
        @functools.partial(pl.kernel, out_type=outs[0], mesh=_mesh,
                           compiler_params=_sc_params, scratch_types=scratch)
        def scatter(msg_hbm, dst_hbm, zeros_hbm, agg_out,
                    dst_v, msg_v, agg_sh):
            cid = lax.axis_index("c")
            sid = lax.axis_index("s")
            w = sid * NC + cid
            pltpu.sync_copy(zeros_hbm, agg_sh.at[pl.ds(sid * RPT, RPT)])
            pltpu.sync_copy(dst_hbm.at[w], dst_v)
            pltpu.sync_copy(msg_hbm.at[pl.ds(w * EPW, EPW)], msg_v)
            plsc.subcore_barrier()

            def body(j, carry):
                pltpu.sync_copy(msg_v.at[pl.ds(j * CH, CH)],
                                agg_sh.at[dst_v.at[j]], add=True)
                return carry

            lax.fori_loop(0, NCHUNK, body, 0)
            plsc.subcore_barrier()
            pltpu.sync_copy(agg_sh.at[pl.ds(sid * RPT, RPT)],
                            agg_out.at[cid, sid])

    return scatter


_scatter_cnt = _make_scatter(True)
_scatter = _make_scatter(False)


BE1 = 640
BE2 = 2000


def _msg_body(in_dim, be, lowp, attr_ref, xs_ref, Ap_ref, bp_ref, tile_ref,
              sel_ref, out_ref):
    w = jnp.dot(attr_ref[...], Ap_ref[...],
                preferred_element_type=jnp.float32) + bp_ref[...]
    w = jnp.maximum(w, 0.0)
    if in_dim >= 128:
        xs_t = jnp.concatenate([xs_ref[...]] * H, axis=1)
    else:
        xs_t = jnp.dot(xs_ref[...], tile_ref[...],
                       preferred_element_type=jnp.float32)
    prod = w * xs_t
    if lowp:
        prod = prod.astype(jnp.bfloat16)
    out_ref[...] = jnp.dot(prod, sel_ref[...],
                           preferred_element_type=jnp.float32)


def _make_msg(in_dim, be, lowp):
    k = in_dim * H
    grid = E // be
    return pl.pallas_call(
        functools.partial(_msg_body, in_dim, be, lowp),
        grid=(grid,),
        in_specs=[
            pl.BlockSpec((be, EDGE_IN), lambda i: (i, 0)),
            pl.BlockSpec((be, in_dim), lambda i: (i, 0)),
            pl.BlockSpec((EDGE_IN, k), lambda i: (0, 0)),
            pl.BlockSpec((1, k), lambda i: (0, 0)),
            pl.BlockSpec((in_dim, k), lambda i: (0, 0)),
            pl.BlockSpec((k, H), lambda i: (0, 0)),
        ],
        out_specs=pl.BlockSpec((be, H), lambda i: (i, 0)),
        out_shape=jax.ShapeDtypeStruct((E, H), jnp.float32),
    )


_msg1 = _make_msg(NODE_IN, BE1, True)
_msg23 = _make_msg(H, BE2, False)


def _update_body(agg_ref, cnt_ref, x_ref, root_ref, bias_ref, out_ref):
    agg = agg_ref[0] + agg_ref[1]
    cnt = jnp.maximum(cnt_ref[0] + cnt_ref[1], 1.0)
    out_ref[...] = jnp.maximum(
        agg / cnt + jnp.dot(x_ref[...], root_ref[...],
                            preferred_element_type=jnp.float32) + bias_ref[...],
        0.0)


def _make_update(in_dim):
    return pl.pallas_call(
        _update_body,
        out_shape=jax.ShapeDtypeStruct((N, H), jnp.float32),
    )


_update128 = _make_update(NODE_IN)
_update16 = _make_update(H)


def _final_body(agg_ref, cnt_ref, x_ref, root_ref, bias_ref,
                Wo1_ref, bo1_ref, Wo2_ref, bo2_ref, out_ref):
    agg = agg_ref[0] + agg_ref[1]
    cnt = jnp.maximum(cnt_ref[0] + cnt_ref[1], 1.0)
    h3 = jnp.maximum(
        agg / cnt + jnp.dot(x_ref[...], root_ref[...],
                            preferred_element_type=jnp.float32) + bias_ref[...],
        0.0)
    h4 = jnp.maximum(jnp.dot(h3, Wo1_ref[...],
                             preferred_element_type=jnp.float32) + bo1_ref[...],
                     0.0)
    out_ref[...] = (jnp.dot(h4, Wo2_ref[...],
                            preferred_element_type=jnp.float32) + bo2_ref[...])


_final = pl.pallas_call(
    _final_body,
    out_shape=jax.ShapeDtypeStruct((N, 1), jnp.float32),
)


def _perm(A, b, in_dim):
    Ap = A.reshape(EDGE_IN, in_dim, H).transpose(0, 2, 1).reshape(EDGE_IN, in_dim * H)
    bp = b.reshape(in_dim, H).T.reshape(1, in_dim * H)
    return Ap, bp


def _sel(in_dim):
    k = in_dim * H
    rows = jnp.arange(k) // in_dim
    return rows[:, None] == jnp.arange(H)[None, :]


def _tilemat(in_dim):
    k = in_dim * H
    cols = jnp.arange(k) % in_dim
    return (jnp.arange(in_dim)[:, None] == cols[None, :]).astype(jnp.float32)


def kernel(x, edge_index, edge_attr,
           A1, b1, root1, bias1,
           A2, b2, root2, bias2,
           A3, b3, root3, bias3,
           Wo1, bo1, Wo2, bo2):
    src = edge_index[0].reshape(NW, NCHUNK, CH)
    dst = edge_index[1].reshape(NW, NCHUNK, CH)
    zeros = jnp.zeros((RPT, H), jnp.float32)
    ones = jnp.ones((CH, H), jnp.float32)

    A1p, b1p = _perm(A1, b1, NODE_IN)
    A2p, b2p = _perm(A2, b2, H)
    A3p, b3p = _perm(A3, b3, H)
    sel1 = _sel(NODE_IN).astype(jnp.bfloat16)
    sel2 = _sel(H).astype(jnp.float32)
    t1 = _tilemat(NODE_IN)
    t2 = _tilemat(H)

    xs = _gather128(x, src)
    msg = _msg1(edge_attr, xs, A1p, b1p, t1, sel1)
    agg, cnt = _scatter_cnt(msg, dst, zeros, ones)
    agg = agg.reshape(NC, N, H)
    cnt = cnt.reshape(NC, N, H)
    h = _update128(agg, cnt, x, root1, bias1.reshape(1, H))

    xs = _gather16(h, src)
    msg = _msg23(edge_attr, xs, A2p, b2p, t2, sel2)
    agg = _scatter(msg, dst, zeros).reshape(NC, N, H)
    h = _update16(agg, cnt, h, root2, bias2.reshape(1, H))

    xs = _gather16(h, src)
    msg = _msg23(edge_attr, xs, A3p, b3p, t2, sel2)
    agg = _scatter(msg, dst, zeros).reshape(NC, N, H)
    out = _final(agg, cnt, h, root3, bias3.reshape(1, H),
                 Wo1, bo1.reshape(1, H), Wo2, bo2.reshape(1, 1))
    return out


# speedup vs baseline: 3.0670x; 1.2014x over previous
import functools

import jax
import jax.numpy as jnp
from jax import lax
from jax.experimental import pallas as pl
from jax.experimental.pallas import tpu as pltpu
from jax.experimental.pallas import tpu_sc as plsc

N = 10000
E = 160000
NODE_IN = 128
EDGE_IN = 4
H = 16

NC = 2
NS = 16
NW = NC * NS
CH = 128
CPW = 40
EPW = CPW * CH
E_PAD = NW * EPW
NSH = N + 240
RPT = NSH // NS

_sc_params = pltpu.CompilerParams(use_tc_tiling_on_sc=False)


@functools.lru_cache(maxsize=1)
def _mesh():
    return plsc.VectorSubcoreMesh(core_axis_name="c", subcore_axis_name="s",
                                  num_cores=NC, num_subcores=NS)


def _wid():
    return lax.axis_index("s") * NC + lax.axis_index("c")


@functools.lru_cache(maxsize=None)
def _make_gather(D):
    @functools.partial(
        pl.kernel,
        out_type=jax.ShapeDtypeStruct((E_PAD, D), jnp.float32),
        mesh=_mesh(),
        compiler_params=_sc_params,
        scratch_types=[
            pltpu.VMEM((CPW, CH), jnp.int32),
            pltpu.VMEM((CH, D), jnp.float32),
            pltpu.SemaphoreType.DMA,
        ],
    )
    def gather(table_hbm, idx_hbm, out_hbm, idx_v, rows_v, sem):
        w = _wid()
        pltpu.sync_copy(idx_hbm.at[pl.ds(w * CPW, CPW)], idx_v)

        def body(j, carry):
            pltpu.async_copy(table_hbm.at[idx_v.at[j]], rows_v, sem).wait()
            pltpu.sync_copy(rows_v, out_hbm.at[pl.ds(w * EPW + j * CH, CH)])
            return carry

        lax.fori_loop(0, CPW, body, 0)

    return gather


@functools.lru_cache(maxsize=None)
def _make_scatter(with_cnt):
    outs = [jax.ShapeDtypeStruct((NC, NSH, H), jnp.float32)]
    scratch = [
        pltpu.VMEM((CPW, CH), jnp.int32),
        pltpu.VMEM((EPW, H), jnp.float32),
        pltpu.VMEM_SHARED((NSH, H), jnp.float32),
    ]
    if with_cnt:
        outs.append(jax.ShapeDtypeStruct((NC, NSH, H), jnp.float32))
        scratch += [
            pltpu.VMEM((CH, H), jnp.float32),
            pltpu.VMEM_SHARED((NSH, H), jnp.float32),
        ]

    if with_cnt:
        @functools.partial(pl.kernel, out_type=tuple(outs), mesh=_mesh(),
                           compiler_params=_sc_params, scratch_types=scratch)
        def scatter(msg_hbm, dst_hbm, zeros_hbm, ones_hbm, agg_out, cnt_out,
                    dst_v, msg_v, agg_sh, ones_v, cnt_sh):
            cid = lax.axis_index("c")
            sid = lax.axis_index("s")
            w = sid * NC + cid
            pltpu.sync_copy(zeros_hbm, agg_sh.at[pl.ds(sid * RPT, RPT)])
            pltpu.sync_copy(zeros_hbm, cnt_sh.at[pl.ds(sid * RPT, RPT)])
            pltpu.sync_copy(ones_hbm, ones_v)
            pltpu.sync_copy(dst_hbm.at[pl.ds(w * CPW, CPW)], dst_v)
            pltpu.sync_copy(msg_hbm.at[pl.ds(w * EPW, EPW)], msg_v)
            plsc.subcore_barrier()

            def body(j, carry):
                pltpu.sync_copy(msg_v.at[pl.ds(j * CH, CH)],
                                agg_sh.at[dst_v.at[j]], add=True)
                pltpu.sync_copy(ones_v, cnt_sh.at[dst_v.at[j]], add=True)
                return carry

            lax.fori_loop(0, CPW, body, 0)
            plsc.subcore_barrier()
            pltpu.sync_copy(agg_sh.at[pl.ds(sid * RPT, RPT)],
                            agg_out.at[cid, pl.ds(sid * RPT, RPT)])
            pltpu.sync_copy(cnt_sh.at[pl.ds(sid * RPT, RPT)],
                            cnt_out.at[cid, pl.ds(sid * RPT, RPT)])
    else:
        @functools.partial(pl.kernel, out_type=outs[0], mesh=_mesh(),
                           compiler_params=_sc_params, scratch_types=scratch)
        def scatter(msg_hbm, dst_hbm, zeros_hbm, agg_out,
                    dst_v, msg_v, agg_sh):
            cid = lax.axis_index("c")
            sid = lax.axis_index("s")
            w = sid * NC + cid
            pltpu.sync_copy(zeros_hbm, agg_sh.at[pl.ds(sid * RPT, RPT)])
            pltpu.sync_copy(dst_hbm.at[pl.ds(w * CPW, CPW)], dst_v)
            pltpu.sync_copy(msg_hbm.at[pl.ds(w * EPW, EPW)], msg_v)
            plsc.subcore_barrier()

            def body(j, carry):
                pltpu.sync_copy(msg_v.at[pl.ds(j * CH, CH)],
                                agg_sh.at[dst_v.at[j]], add=True)
                return carry

            lax.fori_loop(0, CPW, body, 0)
            plsc.subcore_barrier()
            pltpu.sync_copy(agg_sh.at[pl.ds(sid * RPT, RPT)],
                            agg_out.at[cid, pl.ds(sid * RPT, RPT)])

    return scatter


BE1 = 640
BE2 = 3200


def _msg_body(in_dim, be, lowp, attr_ref, xs_ref, Ap_ref, bp_ref, tile_ref,
              sel_ref, out_ref):
    sub = be // 8
    w = jnp.dot(attr_ref[...], Ap_ref[...],
                preferred_element_type=jnp.float32) + bp_ref[...]
    w = jnp.maximum(w, 0.0)
    if in_dim >= 128:
        xs = xs_ref[...]
        xs_t = jnp.concatenate([xs] * H, axis=1)
    else:
        xw = xs_ref[...]
        xs = jnp.concatenate(
            [xw[:, k * in_dim:(k + 1) * in_dim] for k in range(8)], axis=0)
        xs_t = jnp.dot(xs, tile_ref[...], preferred_element_type=jnp.float32)
    prod = w * xs_t
    if lowp:
        prod = prod.astype(jnp.bfloat16)
    msg = jnp.dot(prod, sel_ref[...], preferred_element_type=jnp.float32)
    out_ref[...] = jnp.concatenate(
        [msg[k * sub:(k + 1) * sub, :] for k in range(8)], axis=1)


def _make_msg(in_dim, be, lowp):
    k = in_dim * H
    grid = E // be
    obr = be * H // 128
    if in_dim >= 128:
        xs_spec = pl.BlockSpec((be, in_dim), lambda i: (i, 0))
    else:
        xs_spec = pl.BlockSpec((be * in_dim // 128, 128), lambda i: (i, 0))
    return pl.pallas_call(
        functools.partial(_msg_body, in_dim, be, lowp),
        grid=(grid,),
        in_specs=[
            pl.BlockSpec((be, EDGE_IN), lambda i: (i, 0)),
            xs_spec,
            pl.BlockSpec((EDGE_IN, k), lambda i: (0, 0)),
            pl.BlockSpec((1, k), lambda i: (0, 0)),
            pl.BlockSpec((in_dim, k), lambda i: (0, 0)),
            pl.BlockSpec((k, H), lambda i: (0, 0)),
        ],
        out_specs=pl.BlockSpec((obr, 128), lambda i: (i, 0)),
        out_shape=jax.ShapeDtypeStruct((E_PAD * H // 128, 128), jnp.float32),
    )


_msg1 = _make_msg(NODE_IN, BE1, True)
_msg23 = _make_msg(H, BE2, False)


def _update_body(in_dim, head, agg_ref, cnt_ref, x_ref, root_ref, bias_ref,
                 *rest):
    if head:
        Wo1_ref, bo1_ref, Wo2_ref, bo2_ref, out_ref = rest
    else:
        (out_ref,) = rest
    agg = (agg_ref[0] + agg_ref[1])[:N]
    cnt = (cnt_ref[0] + cnt_ref[1])[:N]
    cnt = jnp.maximum(cnt, 1.0)
    x = x_ref[...]
    h = jnp.maximum(
        agg / cnt + jnp.dot(x, root_ref[...],
                            preferred_element_type=jnp.float32) + bias_ref[...],
        0.0)
    if head:
        h4 = jnp.maximum(
            jnp.dot(h, Wo1_ref[...],
                    preferred_element_type=jnp.float32) + bo1_ref[...], 0.0)
        out_ref[...] = (jnp.dot(h4, Wo2_ref[...],
                                preferred_element_type=jnp.float32)
                        + bo2_ref[...])
    else:
        out_ref[...] = h


def _make_update(in_dim):
    return pl.pallas_call(
        functools.partial(_update_body, in_dim, False),
        out_shape=jax.ShapeDtypeStruct((N, H), jnp.float32),
    )


_update128 = _make_update(NODE_IN)
_update16 = _make_update(H)

_final = pl.pallas_call(
    functools.partial(_update_body, H, True),
    out_shape=jax.ShapeDtypeStruct((N, 1), jnp.float32),
)


def _perm(A, b, in_dim):
    Ap = A.reshape(EDGE_IN, in_dim, H).transpose(0, 2, 1).reshape(EDGE_IN, in_dim * H)
    bp = b.reshape(in_dim, H).T.reshape(1, in_dim * H)
    return Ap, bp


def _sel(in_dim):
    k = in_dim * H
    rows = jnp.arange(k) // in_dim
    return rows[:, None] == jnp.arange(H)[None, :]


def _tilemat(in_dim):
    k = in_dim * H
    cols = jnp.arange(k) % in_dim
    return (jnp.arange(in_dim)[:, None] == cols[None, :]).astype(jnp.float32)


def kernel(x, edge_index, edge_attr,
           A1, b1, root1, bias1,
           A2, b2, root2, bias2,
           A3, b3, root3, bias3,
           Wo1, bo1, Wo2, bo2):
    pad = E_PAD - E
    src = jnp.concatenate(
        [edge_index[0], jnp.zeros((pad,), jnp.int32)]).reshape(E_PAD // CH, CH)
    dst = jnp.concatenate(
        [edge_index[1], jnp.full((pad,), N, jnp.int32)]).reshape(E_PAD // CH, CH)
    zeros = jnp.zeros((RPT, H), jnp.float32)
    ones = jnp.ones((CH, H), jnp.float32)

    A1p, b1p = _perm(A1, b1, NODE_IN)
    A2p, b2p = _perm(A2, b2, H)
    A3p, b3p = _perm(A3, b3, H)
    sel1 = _sel(NODE_IN).astype(jnp.bfloat16)
    sel2 = _sel(H).astype(jnp.float32)
    t1 = _tilemat(NODE_IN)
    t2 = _tilemat(H)

    def permute(core, pad_val, be):
        sub = be // 8
        p = core.reshape(E // be, 8, sub).swapaxes(1, 2).reshape(E)
        return (jnp.concatenate([p, jnp.full((pad,), pad_val, jnp.int32)])
                .reshape(E_PAD // CH, CH))

    dstp1 = permute(edge_index[1], N, BE1)
    dstp2 = permute(edge_index[1], N, BE2)
    srcp2 = permute(edge_index[0], 0, BE2)

    xs = _make_gather(NODE_IN)(x, src)
    msg = _msg1(edge_attr, xs, A1p, b1p, t1, sel1)
    agg, cnt = _make_scatter(True)(msg.reshape(E_PAD, H), dstp1, zeros, ones)
    h = _update128(agg, cnt, x, root1, bias1.reshape(1, H))

    xs = _make_gather(H)(h, srcp2)
    msg = _msg23(edge_attr, xs.reshape(E_PAD * H // 128, 128), A2p, b2p, t2, sel2)
    agg = _make_scatter(False)(msg.reshape(E_PAD, H), dstp2, zeros)
    h = _update16(agg, cnt, h, root2, bias2.reshape(1, H))

    xs = _make_gather(H)(h, srcp2)
    msg = _msg23(edge_attr, xs.reshape(E_PAD * H // 128, 128), A3p, b3p, t2, sel2)
    agg = _make_scatter(False)(msg.reshape(E_PAD, H), dstp2, zeros)
    out = _final(agg, cnt, h, root3, bias3.reshape(1, H),
                 Wo1, bo1.reshape(1, H), Wo2, bo2.reshape(1, 1))
    return out
